# baseline (speedup 1.0000x reference)
import jax
import jax.numpy as jnp
from jax import lax
from jax.experimental import pallas as pl
from jax.experimental.pallas import tpu as pltpu
from jax.experimental.pallas import tpu_sc as plsc

SENT_LEN = 200
D = 64
NUM_WORKERS = 32
CHUNK = 256
GB = 128
NBUF = 2
LANES = 16


def _sc_body(xflat, table2, postab2, out2, mask_out,
             pos_v, idx_f, pair_v, rows2_v, out_v, mask_v, sem_g, sem_w):
    wid = lax.axis_index("s") * 2 + lax.axis_index("c")

    pltpu.sync_copy(postab2, pos_v)

    tokens_per_worker = xflat.shape[0] // NUM_WORKERS
    n_steps = tokens_per_worker // CHUNK
    tok0 = wid * tokens_per_worker

    def tok_base(s):
        return pl.multiple_of(tok0 + s * CHUNK, CHUNK)

    def load_and_fire(s, b):
        base = tok_base(s)
        for j in range(CHUNK // GB):
            pltpu.sync_copy(xflat.at[pl.ds(base + j * GB, GB)],
                            idx_f[b].at[pl.ds(j * GB, GB)])
        for j in range(CHUNK // GB):
            for c in range(GB // LANES):
                sl = pl.ds(j * GB + c * LANES, LANES)
                pair_v[b][j, pl.ds(c * LANES, LANES)] = (
                    lax.shift_right_logical(idx_f[b][sl], jnp.int32(1)))
        for j in range(CHUNK // GB):
            pltpu.async_copy(table2.at[pair_v[b].at[j]],
                             rows2_v[b].at[pl.ds(j * GB, GB)],
                             sem_g[b])

    def wait_gathers(b):
        for j in range(CHUNK // GB):
            pltpu.make_async_copy(
                table2.at[pair_v[b].at[j]],
                rows2_v[b].at[pl.ds(j * GB, GB)],
                sem_g[b]).wait()

    def wb_descriptor(s, b):
        return pltpu.make_async_copy(
            out_v[b],
            out2.at[pl.ds(pl.multiple_of(tok_base(s) // 2, CHUNK // 2),
                          CHUNK // 2)],
            sem_w[b])

    load_and_fire(0, 0)

    def substep(i, s, b):
        wait_gathers(b)

        def prefetch():
            @pl.when(s >= 1)
            def _drain():
                wb_descriptor(s - 1, 1 - b).wait()
            load_and_fire(s + 1, 1 - b)

        if b == 0:
            prefetch()
        else:
            @pl.when(i < n_steps // 2 - 1)
            def _p():
                prefetch()

        phase = lax.rem(s * CHUNK, SENT_LEN)

        @plsc.parallel_loop(0, CHUNK, 1, unroll=2)
        def add_body(r):
            t = idx_f[b][pl.ds(r, LANES)][0]
            src = lax.shift_left(lax.bitwise_and(t, jnp.int32(1)),
                                 jnp.int32(6))
            pp = lax.rem(phase + r, SENT_LEN)
            prow = lax.shift_right_logical(pp, jnp.int32(1))
            poff = lax.shift_left(lax.bitwise_and(pp, jnp.int32(1)),
                                  jnp.int32(6))
            orow = lax.shift_right_logical(r, jnp.int32(1))
            ooff = lax.shift_left(lax.bitwise_and(r, jnp.int32(1)),
                                  jnp.int32(6))
            for c in range(D // LANES):
                out_v[b][orow, pl.ds(ooff + c * LANES, LANES)] = (
                    rows2_v[b][r, pl.ds(src + c * LANES, LANES)]
                    + pos_v[prow, pl.ds(poff + c * LANES, LANES)])

        for j in range(CHUNK // LANES):
            sl = pl.ds(j * LANES, LANES)
            v = idx_f[b][sl]
            mask_v[b][sl] = jnp.where(v == jnp.int32(0), jnp.int32(1),
                                      jnp.int32(0))

        pltpu.sync_copy(mask_v[b], mask_out.at[pl.ds(tok_base(s), CHUNK)])
        wb_descriptor(s, b).start()

    def iteration(i, carry):
        substep(i, 2 * i, 0)
        substep(i, 2 * i + 1, 1)
        return carry

    lax.fori_loop(0, n_steps // 2, iteration, 0)

    wb_descriptor(n_steps - 2, 0).wait()
    wb_descriptor(n_steps - 1, 1).wait()


def kernel(x, token_table, pos_table, positions):
    del positions
    B, L = x.shape
    n_tok = B * L
    xflat = x.astype(jnp.int32).reshape(n_tok)
    table2 = token_table.reshape(token_table.shape[0] // 2, 2 * D)
    postab2 = pos_table.reshape(L // 2, 2 * D)

    mesh = plsc.VectorSubcoreMesh(core_axis_name="c", subcore_axis_name="s")
    out2, mask_i32 = pl.kernel(
        _sc_body,
        out_type=[
            jax.ShapeDtypeStruct((n_tok // 2, 2 * D), jnp.float32),
            jax.ShapeDtypeStruct((n_tok,), jnp.int32),
        ],
        mesh=mesh,
        scratch_types=[
            pltpu.VMEM((L // 2, 2 * D), jnp.float32),
            [pltpu.VMEM((CHUNK + LANES,), jnp.int32)
             for _ in range(NBUF)],
            [pltpu.VMEM((CHUNK // GB, GB), jnp.int32)
             for _ in range(NBUF)],
            [pltpu.VMEM((CHUNK, 2 * D), jnp.float32)
             for _ in range(NBUF)],
            [pltpu.VMEM((CHUNK // 2, 2 * D), jnp.float32)
             for _ in range(NBUF)],
            [pltpu.VMEM((CHUNK,), jnp.int32) for _ in range(NBUF)],
            [pltpu.SemaphoreType.DMA for _ in range(NBUF)],
            [pltpu.SemaphoreType.DMA for _ in range(NBUF)],
        ],
    )(xflat, table2, postab2)

    return (out2.reshape(B, L, D), mask_i32.astype(jnp.bool_).reshape(B, L))


# speedup vs baseline: 1.1519x; 1.1519x over previous
import jax
import jax.numpy as jnp
from jax import lax
from jax.experimental import pallas as pl
from jax.experimental.pallas import tpu as pltpu
from jax.experimental.pallas import tpu_sc as plsc

SENT_LEN = 200
D = 64
NUM_WORKERS = 32
CHUNK = 160
GB = 80
NBUF = 2
LANES = 16


def _sc_body(xflat, table2, postab2, out2, mask_out,
             pos_v, idx_f, pair_v, rows2_v, out_v, mask_v, sem_g, sem_w):
    wid = lax.axis_index("s") * 2 + lax.axis_index("c")

    pltpu.sync_copy(postab2, pos_v)

    tokens_per_worker = xflat.shape[0] // NUM_WORKERS
    n_steps = tokens_per_worker // CHUNK
    tok0 = wid * tokens_per_worker

    def tok_base(s):
        return pl.multiple_of(tok0 + s * CHUNK, CHUNK)

    def load_and_fire(s, b):
        base = tok_base(s)
        for j in range(CHUNK // GB):
            pltpu.sync_copy(xflat.at[pl.ds(base + j * GB, GB)],
                            idx_f[b].at[pl.ds(j * GB, GB)])
        for j in range(CHUNK // GB):
            for c in range(GB // LANES):
                sl = pl.ds(j * GB + c * LANES, LANES)
                pair_v[b][j, pl.ds(c * LANES, LANES)] = (
                    lax.shift_right_logical(idx_f[b][sl], jnp.int32(1)))
        for j in range(CHUNK // GB):
            pltpu.async_copy(table2.at[pair_v[b].at[j]],
                             rows2_v[b].at[pl.ds(j * GB, GB)],
                             sem_g[b])

    def wait_gathers(b):
        for j in range(CHUNK // GB):
            pltpu.make_async_copy(
                table2.at[pair_v[b].at[j]],
                rows2_v[b].at[pl.ds(j * GB, GB)],
                sem_g[b]).wait()

    def wb_descriptor(s, b):
        return pltpu.make_async_copy(
            out_v[b], out2.at[pl.ds(tok_base(s), CHUNK)], sem_w[b])

    load_and_fire(0, 0)

    def substep(i, s, b):
        wait_gathers(b)

        def prefetch():
            @pl.when(s >= 1)
            def _drain():
                wb_descriptor(s - 1, 1 - b).wait()
            load_and_fire(s + 1, 1 - b)

        if b == 0:
            prefetch()
        else:
            @pl.when(i < n_steps // 2 - 1)
            def _p():
                prefetch()

        phase = lax.rem(s * CHUNK, SENT_LEN)

        @plsc.parallel_loop(0, CHUNK, 1, unroll=2)
        def add_body(r):
            t = idx_f[b][pl.ds(r, LANES)][0]
            src = lax.shift_left(lax.bitwise_and(t, jnp.int32(1)),
                                 jnp.int32(6))
            pp = lax.rem(phase + r, SENT_LEN)
            prow = lax.shift_right_logical(pp, jnp.int32(1))
            poff = lax.shift_left(lax.bitwise_and(pp, jnp.int32(1)),
                                  jnp.int32(6))
            for c in range(D // LANES):
                out_v[b][r, pl.ds(c * LANES, LANES)] = (
                    rows2_v[b][r, pl.ds(src + c * LANES, LANES)]
                    + pos_v[prow, pl.ds(poff + c * LANES, LANES)])

        for j in range(CHUNK // LANES):
            sl = pl.ds(j * LANES, LANES)
            v = idx_f[b][sl]
            mask_v[b][sl] = jnp.where(v == jnp.int32(0), jnp.int32(1),
                                      jnp.int32(0))

        pltpu.sync_copy(mask_v[b], mask_out.at[pl.ds(tok_base(s), CHUNK)])
        wb_descriptor(s, b).start()

    def iteration(i, carry):
        substep(i, 2 * i, 0)
        substep(i, 2 * i + 1, 1)
        return carry

    lax.fori_loop(0, n_steps // 2, iteration, 0)

    wb_descriptor(n_steps - 2, 0).wait()
    wb_descriptor(n_steps - 1, 1).wait()


def kernel(x, token_table, pos_table, positions):
    del positions
    B, L = x.shape
    n_tok = B * L
    xflat = x.astype(jnp.int32).reshape(n_tok)
    table2 = token_table.reshape(token_table.shape[0] // 2, 2 * D)
    postab2 = pos_table.reshape(L // 2, 2 * D)

    mesh = plsc.VectorSubcoreMesh(core_axis_name="c", subcore_axis_name="s")
    out2, mask_i32 = pl.kernel(
        _sc_body,
        out_type=[
            jax.ShapeDtypeStruct((n_tok, D), jnp.float32),
            jax.ShapeDtypeStruct((n_tok,), jnp.int32),
        ],
        mesh=mesh,
        scratch_types=[
            pltpu.VMEM((L // 2, 2 * D), jnp.float32),
            [pltpu.VMEM((CHUNK + LANES,), jnp.int32)
             for _ in range(NBUF)],
            [pltpu.VMEM((CHUNK // GB, GB), jnp.int32)
             for _ in range(NBUF)],
            [pltpu.VMEM((CHUNK, 2 * D), jnp.float32)
             for _ in range(NBUF)],
            [pltpu.VMEM((CHUNK, D), jnp.float32)
             for _ in range(NBUF)],
            [pltpu.VMEM((CHUNK,), jnp.int32) for _ in range(NBUF)],
            [pltpu.SemaphoreType.DMA for _ in range(NBUF)],
            [pltpu.SemaphoreType.DMA for _ in range(NBUF)],
        ],
    )(xflat, table2, postab2)

    return (out2.reshape(B, L, D), mask_i32.astype(jnp.bool_).reshape(B, L))


# speedup vs baseline: 1.2420x; 1.0783x over previous
import jax
import jax.numpy as jnp
from jax import lax
from jax.experimental import pallas as pl
from jax.experimental.pallas import tpu as pltpu
from jax.experimental.pallas import tpu_sc as plsc

SENT_LEN = 200
D = 64
NUM_WORKERS = 32
CHUNK = 200
GB = 40
NBUF = 2
LANES = 16


def _sc_body(xflat, table2, postab2, out2, mask_out,
             pos_v, idx_f, pair_v, rows2_v, out_v, mask_v, sem_g, sem_w):
    wid = lax.axis_index("s") * 2 + lax.axis_index("c")

    pltpu.sync_copy(postab2, pos_v)

    tokens_per_worker = xflat.shape[0] // NUM_WORKERS
    n_steps = tokens_per_worker // CHUNK
    tok0 = wid * tokens_per_worker

    def tok_base(s):
        return pl.multiple_of(tok0 + s * CHUNK, CHUNK)

    _pair_starts = [0, LANES, GB - LANES]

    def load_and_fire(s, b):
        pltpu.sync_copy(xflat.at[pl.ds(tok_base(s), CHUNK)],
                        idx_f[b].at[pl.ds(0, CHUNK)])
        for j in range(CHUNK // GB):
            for st in _pair_starts:
                pair_v[b][j, pl.ds(st, LANES)] = lax.shift_right_logical(
                    idx_f[b][pl.ds(j * GB + st, LANES)], jnp.int32(1))
        for j in range(CHUNK // GB):
            pltpu.async_copy(table2.at[pair_v[b].at[j]],
                             rows2_v[b].at[pl.ds(j * GB, GB)],
                             sem_g[b])

    def wait_gathers(b):
        for j in range(CHUNK // GB):
            pltpu.make_async_copy(
                table2.at[pair_v[b].at[j]],
                rows2_v[b].at[pl.ds(j * GB, GB)],
                sem_g[b]).wait()

    def wb_descriptor(s, b):
        return pltpu.make_async_copy(
            out_v[b], out2.at[pl.ds(tok_base(s), CHUNK)], sem_w[b])

    load_and_fire(0, 0)

    def substep(i, s, b):
        wait_gathers(b)

        def prefetch():
            @pl.when(s >= 1)
            def _drain():
                wb_descriptor(s - 1, 1 - b).wait()
            load_and_fire(s + 1, 1 - b)

        if b == 0:
            prefetch()
        else:
            @pl.when(i < n_steps // 2 - 1)
            def _p():
                prefetch()

        @plsc.parallel_loop(0, CHUNK, 1, unroll=2)
        def add_body(r):
            t = idx_f[b][pl.ds(r, LANES)][0]
            src = lax.shift_left(lax.bitwise_and(t, jnp.int32(1)),
                                 jnp.int32(6))
            prow = lax.shift_right_logical(r, jnp.int32(1))
            poff = lax.shift_left(lax.bitwise_and(r, jnp.int32(1)),
                                  jnp.int32(6))
            for c in range(D // LANES):
                out_v[b][r, pl.ds(c * LANES, LANES)] = (
                    rows2_v[b][r, pl.ds(src + c * LANES, LANES)]
                    + pos_v[prow, pl.ds(poff + c * LANES, LANES)])

        mask_starts = [j * LANES for j in range(CHUNK // LANES)]
        if CHUNK % LANES:
            mask_starts.append(CHUNK - LANES)
        for st in mask_starts:
            sl = pl.ds(st, LANES)
            v = idx_f[b][sl]
            mask_v[b][sl] = jnp.where(v == jnp.int32(0), jnp.int32(1),
                                      jnp.int32(0))

        pltpu.sync_copy(mask_v[b], mask_out.at[pl.ds(tok_base(s), CHUNK)])
        wb_descriptor(s, b).start()

    def iteration(i, carry):
        substep(i, 2 * i, 0)
        substep(i, 2 * i + 1, 1)
        return carry

    lax.fori_loop(0, n_steps // 2, iteration, 0)

    wb_descriptor(n_steps - 2, 0).wait()
    wb_descriptor(n_steps - 1, 1).wait()


def kernel(x, token_table, pos_table, positions):
    del positions
    B, L = x.shape
    n_tok = B * L
    xflat = x.astype(jnp.int32).reshape(n_tok)
    table2 = token_table.reshape(token_table.shape[0] // 2, 2 * D)
    postab2 = pos_table.reshape(L // 2, 2 * D)

    mesh = plsc.VectorSubcoreMesh(core_axis_name="c", subcore_axis_name="s")
    out2, mask_i32 = pl.kernel(
        _sc_body,
        out_type=[
            jax.ShapeDtypeStruct((n_tok, D), jnp.float32),
            jax.ShapeDtypeStruct((n_tok,), jnp.int32),
        ],
        mesh=mesh,
        scratch_types=[
            pltpu.VMEM((L // 2, 2 * D), jnp.float32),
            [pltpu.VMEM((CHUNK + LANES,), jnp.int32)
             for _ in range(NBUF)],
            [pltpu.VMEM((CHUNK // GB, GB), jnp.int32)
             for _ in range(NBUF)],
            [pltpu.VMEM((CHUNK, 2 * D), jnp.float32)
             for _ in range(NBUF)],
            [pltpu.VMEM((CHUNK, D), jnp.float32)
             for _ in range(NBUF)],
            [pltpu.VMEM((CHUNK,), jnp.int32) for _ in range(NBUF)],
            [pltpu.SemaphoreType.DMA for _ in range(NBUF)],
            [pltpu.SemaphoreType.DMA for _ in range(NBUF)],
        ],
    )(xflat, table2, postab2)

    return (out2.reshape(B, L, D), mask_i32.astype(jnp.bool_).reshape(B, L))


# speedup vs baseline: 1.2440x; 1.0016x over previous
import jax
import jax.numpy as jnp
from jax import lax
from jax.experimental import pallas as pl
from jax.experimental.pallas import tpu as pltpu
from jax.experimental.pallas import tpu_sc as plsc

SENT_LEN = 200
D = 64
NUM_WORKERS = 32
CHUNK = 200
GB = 40
NBUF = 2
LANES = 16


def _sc_body(xflat, table2, postab2, out2, mask_out,
             pos_v, idx_f, pair_v, rows2_v, out_v, mask_v, sem_g, sem_w):
    wid = lax.axis_index("s") * 2 + lax.axis_index("c")

    pltpu.sync_copy(postab2, pos_v)

    tokens_per_worker = xflat.shape[0] // NUM_WORKERS
    n_steps = tokens_per_worker // CHUNK
    tok0 = wid * tokens_per_worker

    def tok_base(s):
        return pl.multiple_of(tok0 + s * CHUNK, CHUNK)

    _pair_starts = [0, LANES, GB - LANES]

    def load_and_fire(s, b):
        pltpu.sync_copy(xflat.at[pl.ds(tok_base(s), CHUNK)],
                        idx_f[b].at[pl.ds(0, CHUNK)])
        for j in range(CHUNK // GB):
            for st in _pair_starts:
                pair_v[b][j, pl.ds(st, LANES)] = lax.shift_right_logical(
                    idx_f[b][pl.ds(j * GB + st, LANES)], jnp.int32(1))
        for j in range(CHUNK // GB):
            pltpu.async_copy(table2.at[pair_v[b].at[j]],
                             rows2_v[b].at[pl.ds(j * GB, GB)],
                             sem_g[b])

    def wait_gathers(b):
        for j in range(CHUNK // GB):
            pltpu.make_async_copy(
                table2.at[pair_v[b].at[j]],
                rows2_v[b].at[pl.ds(j * GB, GB)],
                sem_g[b]).wait()

    def wb_descriptor(s, b):
        return pltpu.make_async_copy(
            out_v[b], out2.at[pl.ds(tok_base(s), CHUNK)], sem_w[b])

    load_and_fire(0, 0)

    def substep(i, s, b):
        wait_gathers(b)

        def prefetch():
            @pl.when(s >= 1)
            def _drain():
                wb_descriptor(s - 1, 1 - b).wait()
            load_and_fire(s + 1, 1 - b)

        if b == 0:
            prefetch()
        else:
            @pl.when(i < n_steps // 2 - 1)
            def _p():
                prefetch()

        @plsc.parallel_loop(0, CHUNK, 1, unroll=4)
        def add_body(r):
            t = idx_f[b][pl.ds(r, LANES)][0]
            src = lax.shift_left(lax.bitwise_and(t, jnp.int32(1)),
                                 jnp.int32(6))
            prow = lax.shift_right_logical(r, jnp.int32(1))
            poff = lax.shift_left(lax.bitwise_and(r, jnp.int32(1)),
                                  jnp.int32(6))
            for c in range(D // LANES):
                out_v[b][r, pl.ds(c * LANES, LANES)] = (
                    rows2_v[b][r, pl.ds(src + c * LANES, LANES)]
                    + pos_v[prow, pl.ds(poff + c * LANES, LANES)])

        mask_starts = [j * LANES for j in range(CHUNK // LANES)]
        if CHUNK % LANES:
            mask_starts.append(CHUNK - LANES)
        for st in mask_starts:
            sl = pl.ds(st, LANES)
            v = idx_f[b][sl]
            mask_v[b][sl] = jnp.where(v == jnp.int32(0), jnp.int32(1),
                                      jnp.int32(0))

        pltpu.sync_copy(mask_v[b], mask_out.at[pl.ds(tok_base(s), CHUNK)])
        wb_descriptor(s, b).start()

    def iteration(i, carry):
        substep(i, 2 * i, 0)
        substep(i, 2 * i + 1, 1)
        return carry

    lax.fori_loop(0, n_steps // 2, iteration, 0)

    wb_descriptor(n_steps - 2, 0).wait()
    wb_descriptor(n_steps - 1, 1).wait()


def kernel(x, token_table, pos_table, positions):
    del positions
    B, L = x.shape
    n_tok = B * L
    xflat = x.astype(jnp.int32).reshape(n_tok)
    table2 = token_table.reshape(token_table.shape[0] // 2, 2 * D)
    postab2 = pos_table.reshape(L // 2, 2 * D)

    mesh = plsc.VectorSubcoreMesh(core_axis_name="c", subcore_axis_name="s")
    out2, mask_i32 = pl.kernel(
        _sc_body,
        out_type=[
            jax.ShapeDtypeStruct((n_tok, D), jnp.float32),
            jax.ShapeDtypeStruct((n_tok,), jnp.int32),
        ],
        mesh=mesh,
        scratch_types=[
            pltpu.VMEM((L // 2, 2 * D), jnp.float32),
            [pltpu.VMEM((CHUNK + LANES,), jnp.int32)
             for _ in range(NBUF)],
            [pltpu.VMEM((CHUNK // GB, GB), jnp.int32)
             for _ in range(NBUF)],
            [pltpu.VMEM((CHUNK, 2 * D), jnp.float32)
             for _ in range(NBUF)],
            [pltpu.VMEM((CHUNK, D), jnp.float32)
             for _ in range(NBUF)],
            [pltpu.VMEM((CHUNK,), jnp.int32) for _ in range(NBUF)],
            [pltpu.SemaphoreType.DMA for _ in range(NBUF)],
            [pltpu.SemaphoreType.DMA for _ in range(NBUF)],
        ],
    )(xflat, table2, postab2)

    return (out2.reshape(B, L, D), mask_i32.astype(jnp.bool_).reshape(B, L))


# speedup vs baseline: 1.3103x; 1.0532x over previous
import jax
import jax.numpy as jnp
from jax import lax
from jax.experimental import pallas as pl
from jax.experimental.pallas import tpu as pltpu
from jax.experimental.pallas import tpu_sc as plsc

SENT_LEN = 200
D = 64
NUM_WORKERS = 32
CHUNK = 200
GB = 40
NBUF = 2
LANES = 16


def _sc_body(xflat, table2, postab2, out2, mask_out,
             pos_v, idx_f, pair_v, rows2_v, out_v, mask_v,
             sem_g, sem_w, sem_m):
    wid = lax.axis_index("s") * 2 + lax.axis_index("c")

    pltpu.sync_copy(postab2, pos_v)

    tokens_per_worker = xflat.shape[0] // NUM_WORKERS
    n_steps = tokens_per_worker // CHUNK
    tok0 = wid * tokens_per_worker

    def tok_base(s):
        return pl.multiple_of(tok0 + s * CHUNK, CHUNK)

    _pair_starts = [0, LANES, GB - LANES]

    def load_stage(s, b):
        pltpu.sync_copy(xflat.at[pl.ds(tok_base(s), CHUNK)],
                        idx_f[b].at[pl.ds(0, CHUNK)])
        for j in range(CHUNK // GB):
            for st in _pair_starts:
                pair_v[b][j, pl.ds(st, LANES)] = lax.shift_right_logical(
                    idx_f[b][pl.ds(j * GB + st, LANES)], jnp.int32(1))

    def fire_gathers(b):
        for j in range(CHUNK // GB):
            pltpu.async_copy(table2.at[pair_v[b].at[j]],
                             rows2_v[b].at[pl.ds(j * GB, GB)],
                             sem_g[b])

    def load_and_fire(s, b):
        load_stage(s, b)
        fire_gathers(b)

    def wait_gathers(b):
        for j in range(CHUNK // GB):
            pltpu.make_async_copy(
                table2.at[pair_v[b].at[j]],
                rows2_v[b].at[pl.ds(j * GB, GB)],
                sem_g[b]).wait()

    def wb_descriptor(s, b):
        return pltpu.make_async_copy(
            out_v[b], out2.at[pl.ds(tok_base(s), CHUNK)], sem_w[b])

    load_and_fire(0, 0)

    def wbm_descriptor(s, b):
        return pltpu.make_async_copy(
            mask_v[b], mask_out.at[pl.ds(tok_base(s), CHUNK)], sem_m[b])

    def substep(i, s, b):
        def stage():
            load_stage(s + 1, 1 - b)

        def fire():
            @pl.when(s >= 1)
            def _drain():
                wb_descriptor(s - 1, 1 - b).wait()
                wbm_descriptor(s - 1, 1 - b).wait()
            fire_gathers(1 - b)

        if b == 0:
            stage()
            wait_gathers(b)
            fire()
        else:
            @pl.when(i < n_steps // 2 - 1)
            def _s():
                stage()
            wait_gathers(b)

            @pl.when(i < n_steps // 2 - 1)
            def _f():
                fire()

        @plsc.parallel_loop(0, CHUNK, 1, unroll=4)
        def add_body(r):
            t = idx_f[b][pl.ds(r, LANES)][0]
            src = lax.shift_left(lax.bitwise_and(t, jnp.int32(1)),
                                 jnp.int32(6))
            prow = lax.shift_right_logical(r, jnp.int32(1))
            poff = lax.shift_left(lax.bitwise_and(r, jnp.int32(1)),
                                  jnp.int32(6))
            for c in range(D // LANES):
                out_v[b][r, pl.ds(c * LANES, LANES)] = (
                    rows2_v[b][r, pl.ds(src + c * LANES, LANES)]
                    + pos_v[prow, pl.ds(poff + c * LANES, LANES)])

        mask_starts = [j * LANES for j in range(CHUNK // LANES)]
        if CHUNK % LANES:
            mask_starts.append(CHUNK - LANES)
        for st in mask_starts:
            sl = pl.ds(st, LANES)
            v = idx_f[b][sl]
            mask_v[b][sl] = jnp.where(v == jnp.int32(0), jnp.int32(1),
                                      jnp.int32(0))

        wbm_descriptor(s, b).start()
        wb_descriptor(s, b).start()

    def iteration(i, carry):
        substep(i, 2 * i, 0)
        substep(i, 2 * i + 1, 1)
        return carry

    lax.fori_loop(0, n_steps // 2, iteration, 0)

    wb_descriptor(n_steps - 2, 0).wait()
    wbm_descriptor(n_steps - 2, 0).wait()
    wb_descriptor(n_steps - 1, 1).wait()
    wbm_descriptor(n_steps - 1, 1).wait()


def kernel(x, token_table, pos_table, positions):
    del positions
    B, L = x.shape
    n_tok = B * L
    xflat = x.astype(jnp.int32).reshape(n_tok)
    table2 = token_table.reshape(token_table.shape[0] // 2, 2 * D)
    postab2 = pos_table.reshape(L // 2, 2 * D)

    mesh = plsc.VectorSubcoreMesh(core_axis_name="c", subcore_axis_name="s")
    out2, mask_i32 = pl.kernel(
        _sc_body,
        out_type=[
            jax.ShapeDtypeStruct((n_tok, D), jnp.float32),
            jax.ShapeDtypeStruct((n_tok,), jnp.int32),
        ],
        mesh=mesh,
        scratch_types=[
            pltpu.VMEM((L // 2, 2 * D), jnp.float32),
            [pltpu.VMEM((CHUNK + LANES,), jnp.int32)
             for _ in range(NBUF)],
            [pltpu.VMEM((CHUNK // GB, GB), jnp.int32)
             for _ in range(NBUF)],
            [pltpu.VMEM((CHUNK, 2 * D), jnp.float32)
             for _ in range(NBUF)],
            [pltpu.VMEM((CHUNK, D), jnp.float32)
             for _ in range(NBUF)],
            [pltpu.VMEM((CHUNK,), jnp.int32) for _ in range(NBUF)],
            [pltpu.SemaphoreType.DMA for _ in range(NBUF)],
            [pltpu.SemaphoreType.DMA for _ in range(NBUF)],
            [pltpu.SemaphoreType.DMA for _ in range(NBUF)],
        ],
    )(xflat, table2, postab2)

    return (out2.reshape(B, L, D), mask_i32.astype(jnp.bool_).reshape(B, L))
